# Initial kernel scaffold; baseline (speedup 1.0000x reference)
import jax
import jax.numpy as jnp
from jax.experimental import pallas as pl


def kernel(user_idx, user_emb, item_emb, prob_preference, transition_preference):
    raise NotImplementedError("write your pallas kernel here")


# speedup vs baseline: 1.3018x; 1.3018x over previous
import functools

import jax
import jax.numpy as jnp
from jax import lax
from jax.experimental import pallas as pl
from jax.experimental.pallas import tpu as pltpu
from jax.experimental.pallas import tpu_sc as plsc

N_USER = 100000
N_ITEM = 1000
N_MODELS = 4
DIM = 64
BATCH = 4096

try:
    _info = plsc.get_sparse_core_info()
    _NC, _NS = _info.num_cores, _info.num_subcores
except Exception:
    _NC, _NS = 2, 16
_NW = _NC * _NS
_BPW = BATCH // _NW


def _sc_gather(emb_flat, idx_all, pref_cat):
    mesh = plsc.VectorSubcoreMesh(core_axis_name="c", subcore_axis_name="s")

    @functools.partial(
        pl.kernel,
        mesh=mesh,
        out_type=(
            jax.ShapeDtypeStruct((N_MODELS, BATCH, DIM), jnp.float32),
            jax.ShapeDtypeStruct((BATCH, 8), jnp.float32),
        ),
        scratch_types=[
            pltpu.VMEM((_BPW,), jnp.int32),
            pltpu.VMEM((_BPW, DIM), jnp.float32),
            pltpu.VMEM((_BPW, 8), jnp.float32),
            pltpu.SemaphoreType.DMA,
        ],
        compiler_params=pltpu.CompilerParams(use_tc_tiling_on_sc=False),
    )
    def gather_kernel(emb_hbm, idx_hbm, pref_hbm, u_out, p_out,
                      idx_v, rows_v, prow_v, sem):
        wid = lax.axis_index("s") * _NC + lax.axis_index("c")
        base = wid * _BPW
        for m in range(N_MODELS):
            pltpu.sync_copy(idx_hbm.at[m, pl.ds(base, _BPW)], idx_v)
            pltpu.async_copy(emb_hbm.at[idx_v], rows_v, sem).wait()
            pltpu.sync_copy(rows_v, u_out.at[m, pl.ds(base, _BPW)])
        pltpu.sync_copy(idx_hbm.at[0, pl.ds(base, _BPW)], idx_v)
        pltpu.async_copy(pref_hbm.at[idx_v], prow_v, sem).wait()
        pltpu.sync_copy(prow_v, p_out.at[pl.ds(base, _BPW)])

    return gather_kernel(emb_flat, idx_all, pref_cat)


_BB = 512


def _dense_body(pref_ref, u_ref, item_ref, mix_ref, trans_ref):
    pw = jax.nn.softmax(pref_ref[:, 0:N_MODELS], axis=-1)
    tw = jax.nn.softmax(pref_ref[:, N_MODELS:2 * N_MODELS], axis=-1)
    mix = jnp.zeros((_BB, N_ITEM), jnp.float32)
    trans = jnp.zeros((_BB, N_ITEM), jnp.float32)
    for m in range(N_MODELS):
        logits = jnp.dot(u_ref[m], item_ref[m],
                         preferred_element_type=jnp.float32)
        mx = jnp.max(logits, axis=-1, keepdims=True)
        shifted = logits - mx
        ex = jnp.exp(shifted)
        s = jnp.sum(ex, axis=-1, keepdims=True)
        mix = mix + pw[:, m:m + 1] * (shifted - jnp.log(s))
        trans = trans + tw[:, m:m + 1] * (ex / s)
    mix_ref[...] = mix
    trans_ref[...] = trans


def _tc_dense(pref_rows, u_gath, item_t):
    return pl.pallas_call(
        _dense_body,
        grid=(BATCH // _BB,),
        in_specs=[
            pl.BlockSpec((_BB, 2 * N_MODELS), lambda i: (i, 0)),
            pl.BlockSpec((N_MODELS, _BB, DIM), lambda i: (0, i, 0)),
            pl.BlockSpec((N_MODELS, DIM, N_ITEM), lambda i: (0, 0, 0)),
        ],
        out_specs=[
            pl.BlockSpec((_BB, N_ITEM), lambda i: (i, 0)),
            pl.BlockSpec((_BB, N_ITEM), lambda i: (i, 0)),
        ],
        out_shape=[
            jax.ShapeDtypeStruct((BATCH, N_ITEM), jnp.float32),
            jax.ShapeDtypeStruct((BATCH, N_ITEM), jnp.float32),
        ],
    )(pref_rows, u_gath, item_t)


def kernel(user_idx, user_emb, item_emb, prob_preference, transition_preference):
    idx = user_idx.astype(jnp.int32)
    offs = (jnp.arange(N_MODELS, dtype=jnp.int32) * N_USER)[:, None]
    idx_all = idx[None, :] + offs
    emb_flat = user_emb.reshape(N_MODELS * N_USER, DIM)
    pref_cat = jnp.concatenate([prob_preference, transition_preference], axis=1)
    u_gath, pref_rows = _sc_gather(emb_flat, idx_all, pref_cat)
    item_t = jnp.swapaxes(item_emb, 1, 2)
    mix, trans = _tc_dense(pref_rows, u_gath, item_t)
    return (mix, trans)
